# Initial kernel scaffold; baseline (speedup 1.0000x reference)
import jax
import jax.numpy as jnp
from jax.experimental import pallas as pl


def kernel(x, f_ij, idx_i, idx_j, rcut_ij, W_in2f, Wf1, bf1, Wf2, bf2, Wo1, bo1, Wo2, bo2):
    raise NotImplementedError("write your pallas kernel here")


# speedup vs baseline: 2.3589x; 2.3589x over previous
import functools

import jax
import jax.numpy as jnp
from jax import lax
from jax.experimental import pallas as pl
from jax.experimental.pallas import tpu as pltpu
from jax.experimental.pallas import tpu_sc as plsc

_LOG2 = 0.6931471805599453
_NC = 2
_NS = 16
_C = 80
_IB = 25
_NPAD = 10240


def _ssp(v):
    return jnp.maximum(v, 0.0) + jnp.log1p(jnp.exp(-jnp.abs(v))) - _LOG2


def _h_body(x_ref, w_ref, o_ref):
    o_ref[...] = jnp.dot(x_ref[...], w_ref[...],
                         preferred_element_type=jnp.float32)


def _wij_body(f_ref, rc_ref, wf1_ref, bf1_ref, wf2_ref, bf2_ref, o_ref):
    t = jnp.dot(f_ref[...], wf1_ref[...],
                preferred_element_type=jnp.float32) + bf1_ref[...]
    t = _ssp(t)
    t = jnp.dot(t, wf2_ref[...],
                preferred_element_type=jnp.float32) + bf2_ref[...]
    o_ref[...] = t * rc_ref[...]


def _out_body(n, p_ref, wo1_ref, bo1_ref, wo2_ref, bo2_ref, o_ref):
    agg = p_ref[pl.ds(0, n), :] + p_ref[pl.ds(_NPAD, n), :]
    t = _ssp(jnp.dot(agg, wo1_ref[...],
                     preferred_element_type=jnp.float32) + bo1_ref[...])
    o_ref[...] = jnp.dot(t, wo2_ref[...],
                         preferred_element_type=jnp.float32) + bo2_ref[...]


def _sc_body(e, d, h_hbm, wij_hbm, idxi_hbm, idxj_hbm, out_hbm,
             idxi_v, idxj_v, rows_v, wijb_v, agg_sh, gsem):
    c = lax.axis_index("c")
    s = lax.axis_index("s")
    wid = c * _NS + s

    ept = e // (_NC * _NS)
    nchunks = ept // _C
    rows_per_tile = _NPAD // _NS
    nslc = d // 16

    zvec = jnp.zeros((16,), jnp.float32)

    def _zfill(rr, carry):
        for t in range(nslc):
            wijb_v[rr, pl.ds(t * 16, 16)] = zvec
        return carry
    lax.fori_loop(0, _C, _zfill, 0)
    for i in range(rows_per_tile // _C):
        pltpu.sync_copy(wijb_v, agg_sh.at[pl.ds(s * rows_per_tile + i * _C,
                                                _C)])
    plsc.subcore_barrier()

    ebase = wid * ept
    nblk = nchunks // _IB

    def _blk(b, carry):
        pltpu.sync_copy(idxi_hbm.at[wid, b], idxi_v)
        pltpu.sync_copy(idxj_hbm.at[wid, b], idxj_v)

        def _chunk(k, cc2):
            pltpu.sync_copy(
                wij_hbm.at[pl.ds(ebase + (b * _IB + k) * _C, _C)], wijb_v)
            pltpu.async_copy(h_hbm.at[idxj_v.at[k]], rows_v, gsem).wait()

            def _mul(ei, cc):
                for t in range(nslc):
                    sl = pl.ds(t * 16, 16)
                    rows_v[ei, sl] = rows_v[ei, sl] * wijb_v[ei, sl]
                return cc
            lax.fori_loop(0, _C, _mul, 0)

            pltpu.sync_copy(rows_v, agg_sh.at[idxi_v.at[k]], add=True)
            return cc2
        lax.fori_loop(0, _IB, _chunk, 0)
        return carry
    lax.fori_loop(0, nblk, _blk, 0)

    plsc.subcore_barrier()

    r0 = s * rows_per_tile
    pltpu.sync_copy(agg_sh.at[pl.ds(r0, rows_per_tile)],
                    out_hbm.at[pl.ds(c * _NPAD + r0, rows_per_tile)])


def kernel(x, f_ij, idx_i, idx_j, rcut_ij, W_in2f, Wf1, bf1, Wf2, bf2,
           Wo1, bo1, Wo2, bo2):
    n, d = x.shape
    e, r = f_ij.shape
    f = Wf2.shape[1]
    nw = _NC * _NS
    assert e % (nw * _C) == 0 and n <= _NPAD and d % 16 == 0

    h = pl.pallas_call(
        _h_body,
        out_shape=jax.ShapeDtypeStruct((n, f), jnp.float32),
    )(x, W_in2f)

    eb = 8000
    wij = pl.pallas_call(
        _wij_body,
        grid=(e // eb,),
        in_specs=[
            pl.BlockSpec((eb, r), lambda i: (i, 0)),
            pl.BlockSpec((eb, 1), lambda i: (i, 0)),
            pl.BlockSpec((r, f), lambda i: (0, 0)),
            pl.BlockSpec((1, f), lambda i: (0, 0)),
            pl.BlockSpec((f, f), lambda i: (0, 0)),
            pl.BlockSpec((1, f), lambda i: (0, 0)),
        ],
        out_specs=pl.BlockSpec((eb, f), lambda i: (i, 0)),
        out_shape=jax.ShapeDtypeStruct((e, f), jnp.float32),
    )(f_ij, rcut_ij.reshape(e, 1), Wf1, bf1.reshape(1, f),
      Wf2, bf2.reshape(1, f))

    mesh = plsc.VectorSubcoreMesh(core_axis_name="c", subcore_axis_name="s",
                                  num_cores=_NC, num_subcores=_NS)
    nchunks_tile = e // (nw * _C)
    nblk = nchunks_tile // _IB
    assert nchunks_tile % _IB == 0
    sc = pl.kernel(
        functools.partial(_sc_body, e, f),
        out_type=jax.ShapeDtypeStruct((_NC * _NPAD, f), jnp.float32),
        mesh=mesh,
        scratch_types=[
            pltpu.VMEM((_IB, _C), jnp.int32),
            pltpu.VMEM((_IB, _C), jnp.int32),
            pltpu.VMEM((_C, f), jnp.float32),
            pltpu.VMEM((_C, f), jnp.float32),
            pltpu.VMEM_SHARED((_NPAD, f), jnp.float32),
            pltpu.SemaphoreType.DMA,
        ],
    )
    partials = sc(h, wij,
                  idx_i.astype(jnp.int32).reshape(nw, nblk, _IB, _C),
                  idx_j.astype(jnp.int32).reshape(nw, nblk, _IB, _C))

    out = pl.pallas_call(
        functools.partial(_out_body, n),
        out_shape=jax.ShapeDtypeStruct((n, d), jnp.float32),
    )(partials, Wo1, bo1.reshape(1, d), Wo2, bo2.reshape(1, d))
    return out
